# baseline (speedup 1.0000x reference)
import functools

import jax
import jax.numpy as jnp
from jax import lax
from jax.experimental import pallas as pl
from jax.experimental.pallas import tpu as pltpu
from jax.experimental.pallas import tpu_sc as plsc

R, S = 4096, 200
D = 32
V = 1000000
NC, NS = 2, 16
NW = NC * NS
SB, S8 = S // 8, 8
CB, CL = R // 128, 128
D8, DL = D // 8, 8

BC = 512
NBLK = V // BC
TAIL = V - NBLK * BC

_mesh = plsc.VectorSubcoreMesh(core_axis_name="c", subcore_axis_name="s")
_params = pltpu.CompilerParams(use_tc_tiling_on_sc=False,
                               needs_layout_passes=False)
_params_tc = pltpu.CompilerParams(use_tc_tiling_on_sc=True,
                                  needs_layout_passes=False)


@functools.partial(
    pl.kernel,
    out_type=jax.ShapeDtypeStruct((V * D,), jnp.float32),
    mesh=_mesh,
    scratch_types=[
        pltpu.VMEM((D * BC,), jnp.float32),
        pltpu.VMEM((D * BC,), jnp.float32),
        pltpu.VMEM((BC * D,), jnp.float32),
        pltpu.VMEM((BC * D,), jnp.float32),
        pltpu.VMEM((TAIL * D,), jnp.float32),
        pltpu.SemaphoreType.DMA,
        pltpu.SemaphoreType.DMA,
        pltpu.SemaphoreType.DMA,
        pltpu.SemaphoreType.DMA,
    ],
    compiler_params=_params_tc,
)
def _relay2_kernel(tt_hbm, tail_hbm, o_hbm, tb0, tb1, ros0, ros1, tlb,
                   gi0, gi1, wo0, wo1):
    w = lax.axis_index("s") * NC + lax.axis_index("c")
    tb = (tb0, tb1)
    ros = (ros0, ros1)
    gsem = (gi0, gi1)
    wsem = (wo0, wo1)

    iota = lax.iota(jnp.int32, 16)

    def blk_id(t):
        return w + NW * t

    def start_stage(t, slot):
        b = blk_id(t)

        @pl.loop(0, D)
        def _j(j):
            pltpu.async_copy(tt_hbm.at[j, pl.ds(b * BC, BC)],
                             tb[slot].at[pl.ds(j * BC, BC)], gsem[slot])

    def wait_stage(t, slot):
        b = blk_id(t)

        @pl.loop(0, D)
        def _j(j):
            pltpu.make_async_copy(tt_hbm.at[j, pl.ds(b * BC, BC)],
                                  tb[slot].at[pl.ds(j * BC, BC)],
                                  gsem[slot]).wait()

    def transpose_panel(slot):
        for m in range(16):
            perm = (iota + m) & 15
            ra = iota * BC + perm
            wa = perm * D + iota

            @pl.loop(0, BC // 16, unroll=8)
            def _cg(cg):
                c0 = cg * 16
                for j0 in (0, 16):
                    val = plsc.load_gather(tb[slot], [ra + (j0 * BC + c0)])
                    plsc.store_scatter(ros[slot], [wa + (c0 * D + j0)], val)

    def start_write(t, slot):
        b = blk_id(t)
        pltpu.async_copy(ros[slot], o_hbm.at[pl.ds(b * BC * D, BC * D)],
                         wsem[slot])

    def drain_write(slot):
        pltpu.make_async_copy(o_hbm.at[pl.ds(0, BC * D)], ros[slot],
                              wsem[slot]).wait()

    @pl.when(w == 1)
    def _tail():
        pltpu.sync_copy(tail_hbm, tlb)
        pltpu.sync_copy(tlb, o_hbm.at[pl.ds(NBLK * BC * D, TAIL * D)])

    start_stage(0, 0)

    @pl.loop(0, 61, step=2)
    def _blk(t):
        @pl.when(blk_id(t + 1) < NBLK)
        def _():
            start_stage(t + 1, 1)
        wait_stage(t, 0)

        @pl.when(t >= 2)
        def _():
            drain_write(0)
        transpose_panel(0)
        start_write(t, 0)

        @pl.when(blk_id(t + 2) < NBLK)
        def _():
            start_stage(t + 2, 0)

        @pl.when(blk_id(t + 1) < NBLK)
        def _():
            wait_stage(t + 1, 1)

            @pl.when(t >= 2)
            def _():
                drain_write(1)
            transpose_panel(1)
            start_write(t + 1, 1)

    drain_write(0)
    drain_write(1)


@functools.partial(
    pl.kernel,
    out_type=jax.ShapeDtypeStruct((V, D), jnp.float32),
    mesh=_mesh,
    scratch_types=[
        pltpu.VMEM((D, BC), jnp.float32),
        pltpu.VMEM((D, BC), jnp.float32),
        pltpu.VMEM((BC, D + 1), jnp.float32),
        pltpu.VMEM((BC, D + 1), jnp.float32),
        pltpu.VMEM((D, TAIL), jnp.float32),
        pltpu.VMEM((TAIL, D + 1), jnp.float32),
        pltpu.SemaphoreType.DMA,
        pltpu.SemaphoreType.DMA,
        pltpu.SemaphoreType.DMA,
        pltpu.SemaphoreType.DMA,
    ],
    compiler_params=_params,
)
def _relay_kernel(tt_hbm, t_hbm, tb0, tb1, ros0, ros1, tbt, rost,
                  gi0, gi1, wo0, wo1):
    w = lax.axis_index("s") * NC + lax.axis_index("c")
    tb = (tb0, tb1)
    ros = (ros0, ros1)
    gsem = (gi0, gi1)
    wsem = (wo0, wo1)

    iota = lax.iota(jnp.int32, 16)

    def blk_id(t):
        return w + NW * t

    def start_stage(t, slot):
        b = blk_id(t)
        pltpu.async_copy(tt_hbm.at[:, pl.ds(b * BC, BC)], tb[slot],
                         gsem[slot])

    def wait_stage(t, slot):
        b = blk_id(t)
        pltpu.make_async_copy(tt_hbm.at[:, pl.ds(b * BC, BC)], tb[slot],
                              gsem[slot]).wait()

    def transpose_panel(slot):
        @pl.loop(0, D)
        def _feat(j):
            jv = jnp.full((16,), 0, jnp.int32) + j
            for k in range(BC // 16):
                val = tb[slot][j, pl.ds(k * 16, 16)]
                plsc.store_scatter(ros[slot], [iota + 16 * k, jv], val)

    def start_write(t, slot):
        b = blk_id(t)
        pltpu.async_copy(ros[slot].at[:, pl.ds(0, D)],
                         t_hbm.at[pl.ds(b * BC, BC), :], wsem[slot])

    def drain_write(slot):
        pltpu.make_async_copy(t_hbm.at[pl.ds(0, BC), :],
                              ros[slot].at[:, pl.ds(0, D)],
                              wsem[slot]).wait()

    start_stage(0, 0)

    @pl.loop(0, 61, step=2)
    def _blk(t):
        @pl.when(blk_id(t + 1) < NBLK)
        def _():
            start_stage(t + 1, 1)
        wait_stage(t, 0)

        @pl.when(t >= 2)
        def _():
            drain_write(0)
        transpose_panel(0)
        start_write(t, 0)

        @pl.when(blk_id(t + 2) < NBLK)
        def _():
            start_stage(t + 2, 0)

        @pl.when(blk_id(t + 1) < NBLK)
        def _():
            wait_stage(t + 1, 1)

            @pl.when(t >= 2)
            def _():
                drain_write(1)
            transpose_panel(1)
            start_write(t + 1, 1)

    drain_write(0)
    drain_write(1)

    @pl.when(w == 1)
    def _tail():
        pltpu.sync_copy(tt_hbm.at[:, pl.ds(NBLK * BC, TAIL)], tbt)

        @pl.loop(0, D)
        def _feat(j):
            jv = jnp.full((16,), 0, jnp.int32) + j
            for k in range(TAIL // 16):
                val = tbt[j, pl.ds(k * 16, 16)]
                plsc.store_scatter(rost, [iota + 16 * k, jv], val)
        pltpu.sync_copy(rost.at[:, pl.ds(0, D)],
                        t_hbm.at[pl.ds(NBLK * BC, TAIL), :])


@functools.partial(
    pl.kernel,
    out_type=jax.ShapeDtypeStruct((S, D8, CB, DL, CL), jnp.float32),
    mesh=_mesh,
    scratch_types=[
        pltpu.VMEM((SB, S8, CL), jnp.int32),
        pltpu.VMEM((CL, D), jnp.float32),
        pltpu.VMEM((CL, D), jnp.float32),
        pltpu.VMEM((D8, DL, CL + 1), jnp.float32),
        pltpu.VMEM((D8, DL, CL + 1), jnp.float32),
        pltpu.SemaphoreType.DMA,
        pltpu.SemaphoreType.DMA,
        pltpu.SemaphoreType.DMA,
        pltpu.SemaphoreType.DMA,
    ],
    compiler_params=_params,
)
def _enc_kernel(w4_hbm, t_hbm, out_hbm,
                idxall, rows0, rows1, tt0, tt1, g0, g1, ws0, ws1):
    wrb = lax.axis_index("s") * NC + lax.axis_index("c")

    pltpu.sync_copy(w4_hbm.at[:, wrb], idxall)

    rows = (rows0, rows1)
    tt = (tt0, tt1)
    gsem = (g0, g1)
    wsem = (ws0, ws1)

    iota = lax.iota(jnp.int32, 16)
    d8v = (iota // 8, iota // 8 + 2)
    dlv = iota - (iota // 8) * 8

    def start_gather(s, slot):
        sb = s // 8
        s8 = s - 8 * sb
        pltpu.async_copy(t_hbm.at[idxall.at[sb, s8]], rows[slot],
                         gsem[slot])

    def wait_gather(s, slot):
        sb = s // 8
        s8 = s - 8 * sb
        pltpu.make_async_copy(t_hbm.at[idxall.at[sb, s8]], rows[slot],
                              gsem[slot]).wait()

    def transpose_unit(slot):
        @pl.loop(0, CL, unroll=8)
        def _row(rl):
            rlv = jnp.full((16,), 0, jnp.int32) + rl
            for h in range(2):
                val = rows[slot][rl, pl.ds(h * 16, 16)]
                plsc.store_scatter(tt[slot], [d8v[h], dlv, rlv], val)

    def start_write(s, slot):
        pltpu.async_copy(tt[slot].at[:, :, pl.ds(0, CL)],
                         out_hbm.at[s, :, wrb], wsem[slot])

    def drain_write(slot):
        pltpu.make_async_copy(out_hbm.at[0, :, wrb],
                              tt[slot].at[:, :, pl.ds(0, CL)],
                              wsem[slot]).wait()

    start_gather(0, 0)

    @pl.loop(0, S, step=2)
    def _unit(s):
        start_gather(s + 1, 1)
        wait_gather(s, 0)

        @pl.when(s >= 2)
        def _():
            drain_write(0)
        transpose_unit(0)
        start_write(s, 0)

        @pl.when(s + 2 < S)
        def _():
            start_gather(s + 2, 0)
        wait_gather(s + 1, 1)

        @pl.when(s >= 2)
        def _():
            drain_write(1)
        transpose_unit(1)
        start_write(s + 1, 1)

    drain_write(0)
    drain_write(1)


def kernel(words, table):
    w4 = words.T.reshape(SB, S8, CB, CL).transpose(0, 2, 1, 3)
    tail = table[NBLK * BC:].reshape(TAIL * D)
    t_rm = _relay2_kernel(table.T, tail).reshape(V, D)
    out5 = _enc_kernel(w4, t_rm)
    return out5.transpose(2, 4, 0, 1, 3).reshape(R, S, D)


# speedup vs baseline: 1.0647x; 1.0647x over previous
import functools

import jax
import jax.numpy as jnp
from jax import lax
from jax.experimental import pallas as pl
from jax.experimental.pallas import tpu as pltpu
from jax.experimental.pallas import tpu_sc as plsc

R, S = 4096, 200
D = 32
V = 1000000
NC, NS = 2, 16
NW = NC * NS
SB, S8 = S // 8, 8
CB, CL = R // 128, 128
D8, DL = D // 8, 8

BC = 512
NBLK = V // BC
TAIL = V - NBLK * BC

_mesh = plsc.VectorSubcoreMesh(core_axis_name="c", subcore_axis_name="s")
_params = pltpu.CompilerParams(use_tc_tiling_on_sc=False,
                               needs_layout_passes=False)
_params_tc = pltpu.CompilerParams(use_tc_tiling_on_sc=True,
                                  needs_layout_passes=False)


@functools.partial(
    pl.kernel,
    out_type=jax.ShapeDtypeStruct((V * D,), jnp.float32),
    mesh=_mesh,
    scratch_types=[
        pltpu.VMEM((D * BC,), jnp.float32),
        pltpu.VMEM((D * BC,), jnp.float32),
        pltpu.VMEM((BC * D,), jnp.float32),
        pltpu.VMEM((BC * D,), jnp.float32),
        pltpu.VMEM((TAIL * D,), jnp.float32),
        pltpu.SemaphoreType.DMA,
        pltpu.SemaphoreType.DMA,
        pltpu.SemaphoreType.DMA,
        pltpu.SemaphoreType.DMA,
    ],
    compiler_params=_params_tc,
)
def _relay2_kernel(tt_hbm, tail_hbm, o_hbm, tb0, tb1, ros0, ros1, tlb,
                   gi0, gi1, wo0, wo1):
    w = lax.axis_index("s") * NC + lax.axis_index("c")
    tb = (tb0, tb1)
    ros = (ros0, ros1)
    gsem = (gi0, gi1)
    wsem = (wo0, wo1)

    iota = lax.iota(jnp.int32, 16)

    def blk_id(t):
        return w + NW * t

    def start_stage(t, slot):
        b = blk_id(t)

        @pl.loop(0, D)
        def _j(j):
            pltpu.async_copy(tt_hbm.at[j, pl.ds(b * BC, BC)],
                             tb[slot].at[pl.ds(j * BC, BC)], gsem[slot])

    def wait_stage(t, slot):
        b = blk_id(t)

        @pl.loop(0, D)
        def _j(j):
            pltpu.make_async_copy(tt_hbm.at[j, pl.ds(b * BC, BC)],
                                  tb[slot].at[pl.ds(j * BC, BC)],
                                  gsem[slot]).wait()

    def transpose_panel(slot):
        for m in range(16):
            perm = (iota + m) & 15
            ra = iota * BC + perm
            wa = perm * D + iota

            @pl.loop(0, BC // 16, unroll=4)
            def _cg(cg):
                c0 = cg * 16
                for j0 in (0, 16):
                    val = plsc.load_gather(tb[slot], [ra + (j0 * BC + c0)])
                    plsc.store_scatter(ros[slot], [wa + (c0 * D + j0)], val)

    def start_write(t, slot):
        b = blk_id(t)
        pltpu.async_copy(ros[slot], o_hbm.at[pl.ds(b * BC * D, BC * D)],
                         wsem[slot])

    def drain_write(slot):
        pltpu.make_async_copy(o_hbm.at[pl.ds(0, BC * D)], ros[slot],
                              wsem[slot]).wait()

    @pl.when(w == 1)
    def _tail():
        pltpu.sync_copy(tail_hbm, tlb)
        pltpu.sync_copy(tlb, o_hbm.at[pl.ds(NBLK * BC * D, TAIL * D)])

    start_stage(0, 0)

    @pl.loop(0, 61, step=2)
    def _blk(t):
        @pl.when(blk_id(t + 1) < NBLK)
        def _():
            start_stage(t + 1, 1)
        wait_stage(t, 0)

        @pl.when(t >= 2)
        def _():
            drain_write(0)
        transpose_panel(0)
        start_write(t, 0)

        @pl.when(blk_id(t + 2) < NBLK)
        def _():
            start_stage(t + 2, 0)

        @pl.when(blk_id(t + 1) < NBLK)
        def _():
            wait_stage(t + 1, 1)

            @pl.when(t >= 2)
            def _():
                drain_write(1)
            transpose_panel(1)
            start_write(t + 1, 1)

    drain_write(0)
    drain_write(1)


@functools.partial(
    pl.kernel,
    out_type=jax.ShapeDtypeStruct((V, D), jnp.float32),
    mesh=_mesh,
    scratch_types=[
        pltpu.VMEM((D, BC), jnp.float32),
        pltpu.VMEM((D, BC), jnp.float32),
        pltpu.VMEM((BC, D + 1), jnp.float32),
        pltpu.VMEM((BC, D + 1), jnp.float32),
        pltpu.VMEM((D, TAIL), jnp.float32),
        pltpu.VMEM((TAIL, D + 1), jnp.float32),
        pltpu.SemaphoreType.DMA,
        pltpu.SemaphoreType.DMA,
        pltpu.SemaphoreType.DMA,
        pltpu.SemaphoreType.DMA,
    ],
    compiler_params=_params,
)
def _relay_kernel(tt_hbm, t_hbm, tb0, tb1, ros0, ros1, tbt, rost,
                  gi0, gi1, wo0, wo1):
    w = lax.axis_index("s") * NC + lax.axis_index("c")
    tb = (tb0, tb1)
    ros = (ros0, ros1)
    gsem = (gi0, gi1)
    wsem = (wo0, wo1)

    iota = lax.iota(jnp.int32, 16)

    def blk_id(t):
        return w + NW * t

    def start_stage(t, slot):
        b = blk_id(t)
        pltpu.async_copy(tt_hbm.at[:, pl.ds(b * BC, BC)], tb[slot],
                         gsem[slot])

    def wait_stage(t, slot):
        b = blk_id(t)
        pltpu.make_async_copy(tt_hbm.at[:, pl.ds(b * BC, BC)], tb[slot],
                              gsem[slot]).wait()

    def transpose_panel(slot):
        @pl.loop(0, D)
        def _feat(j):
            jv = jnp.full((16,), 0, jnp.int32) + j
            for k in range(BC // 16):
                val = tb[slot][j, pl.ds(k * 16, 16)]
                plsc.store_scatter(ros[slot], [iota + 16 * k, jv], val)

    def start_write(t, slot):
        b = blk_id(t)
        pltpu.async_copy(ros[slot].at[:, pl.ds(0, D)],
                         t_hbm.at[pl.ds(b * BC, BC), :], wsem[slot])

    def drain_write(slot):
        pltpu.make_async_copy(t_hbm.at[pl.ds(0, BC), :],
                              ros[slot].at[:, pl.ds(0, D)],
                              wsem[slot]).wait()

    start_stage(0, 0)

    @pl.loop(0, 61, step=2)
    def _blk(t):
        @pl.when(blk_id(t + 1) < NBLK)
        def _():
            start_stage(t + 1, 1)
        wait_stage(t, 0)

        @pl.when(t >= 2)
        def _():
            drain_write(0)
        transpose_panel(0)
        start_write(t, 0)

        @pl.when(blk_id(t + 2) < NBLK)
        def _():
            start_stage(t + 2, 0)

        @pl.when(blk_id(t + 1) < NBLK)
        def _():
            wait_stage(t + 1, 1)

            @pl.when(t >= 2)
            def _():
                drain_write(1)
            transpose_panel(1)
            start_write(t + 1, 1)

    drain_write(0)
    drain_write(1)

    @pl.when(w == 1)
    def _tail():
        pltpu.sync_copy(tt_hbm.at[:, pl.ds(NBLK * BC, TAIL)], tbt)

        @pl.loop(0, D)
        def _feat(j):
            jv = jnp.full((16,), 0, jnp.int32) + j
            for k in range(TAIL // 16):
                val = tbt[j, pl.ds(k * 16, 16)]
                plsc.store_scatter(rost, [iota + 16 * k, jv], val)
        pltpu.sync_copy(rost.at[:, pl.ds(0, D)],
                        t_hbm.at[pl.ds(NBLK * BC, TAIL), :])


@functools.partial(
    pl.kernel,
    out_type=jax.ShapeDtypeStruct((S, D8, CB, DL, CL), jnp.float32),
    mesh=_mesh,
    scratch_types=[
        pltpu.VMEM((SB, S8, CL), jnp.int32),
        pltpu.VMEM((CL, D), jnp.float32),
        pltpu.VMEM((CL, D), jnp.float32),
        pltpu.VMEM((D8, DL, CL + 1), jnp.float32),
        pltpu.VMEM((D8, DL, CL + 1), jnp.float32),
        pltpu.SemaphoreType.DMA,
        pltpu.SemaphoreType.DMA,
        pltpu.SemaphoreType.DMA,
        pltpu.SemaphoreType.DMA,
    ],
    compiler_params=_params,
)
def _enc_kernel(w4_hbm, t_hbm, out_hbm,
                idxall, rows0, rows1, tt0, tt1, g0, g1, ws0, ws1):
    wrb = lax.axis_index("s") * NC + lax.axis_index("c")

    pltpu.sync_copy(w4_hbm.at[:, wrb], idxall)

    rows = (rows0, rows1)
    tt = (tt0, tt1)
    gsem = (g0, g1)
    wsem = (ws0, ws1)

    iota = lax.iota(jnp.int32, 16)
    d8v = (iota // 8, iota // 8 + 2)
    dlv = iota - (iota // 8) * 8

    def start_gather(s, slot):
        sb = s // 8
        s8 = s - 8 * sb
        pltpu.async_copy(t_hbm.at[idxall.at[sb, s8]], rows[slot],
                         gsem[slot])

    def wait_gather(s, slot):
        sb = s // 8
        s8 = s - 8 * sb
        pltpu.make_async_copy(t_hbm.at[idxall.at[sb, s8]], rows[slot],
                              gsem[slot]).wait()

    def transpose_unit(slot):
        @pl.loop(0, CL, unroll=8)
        def _row(rl):
            rlv = jnp.full((16,), 0, jnp.int32) + rl
            for h in range(2):
                val = rows[slot][rl, pl.ds(h * 16, 16)]
                plsc.store_scatter(tt[slot], [d8v[h], dlv, rlv], val)

    def start_write(s, slot):
        pltpu.async_copy(tt[slot].at[:, :, pl.ds(0, CL)],
                         out_hbm.at[s, :, wrb], wsem[slot])

    def drain_write(slot):
        pltpu.make_async_copy(out_hbm.at[0, :, wrb],
                              tt[slot].at[:, :, pl.ds(0, CL)],
                              wsem[slot]).wait()

    start_gather(0, 0)

    @pl.loop(0, S, step=2)
    def _unit(s):
        start_gather(s + 1, 1)
        wait_gather(s, 0)

        @pl.when(s >= 2)
        def _():
            drain_write(0)
        transpose_unit(0)
        start_write(s, 0)

        @pl.when(s + 2 < S)
        def _():
            start_gather(s + 2, 0)
        wait_gather(s + 1, 1)

        @pl.when(s >= 2)
        def _():
            drain_write(1)
        transpose_unit(1)
        start_write(s + 1, 1)

    drain_write(0)
    drain_write(1)


def kernel(words, table):
    w4 = words.T.reshape(SB, S8, CB, CL).transpose(0, 2, 1, 3)
    tail = table[NBLK * BC:].reshape(TAIL * D)
    t_rm = _relay2_kernel(table.T, tail).reshape(V, D)
    out5 = _enc_kernel(w4, t_rm)
    return out5.transpose(2, 4, 0, 1, 3).reshape(R, S, D)
